# baseline (speedup 1.0000x reference)
import functools

import jax
import jax.numpy as jnp
from jax import lax
from jax.experimental import pallas as pl
from jax.experimental.pallas import tpu as pltpu
from jax.experimental.pallas import tpu_sc as plsc

N = 10000
E = 160000
F = 256
TCLS = 64
TPAD = 128
P_FRAC = 0.5

NC = 2
NS = 16
NW = NC * NS
NPW = 320

ECHUNK = 128
NCHUNKS = E // ECHUNK
CAP = 8192
GMAX = CAP - 16


def _tc_matmul1(x, w, b):
    def body(x_ref, w_ref, b_ref, o_ref):
        o_ref[...] = jnp.tanh(
            jnp.dot(x_ref[...], w_ref[...], preferred_element_type=jnp.float32)
            + b_ref[...]
        )

    return pl.pallas_call(
        body,
        grid=(10,),
        in_specs=[
            pl.BlockSpec((N // 10, 256), lambda i: (i, 0)),
            pl.BlockSpec((256, TPAD), lambda i: (0, 0)),
            pl.BlockSpec((1, TPAD), lambda i: (0, 0)),
        ],
        out_specs=pl.BlockSpec((N // 10, TPAD), lambda i: (i, 0)),
        out_shape=jax.ShapeDtypeStruct((N, TPAD), jnp.float32),
    )(x, w, b.reshape(1, TPAD))


def _tc_matmul2(hrp, x, w, b):
    def body(h_ref, x_ref, w_ref, b_ref, o_ref):
        o_ref[...] = (
            jnp.dot(h_ref[...] + x_ref[...], w_ref[...],
                    preferred_element_type=jnp.float32)
            + b_ref[...]
        )

    return pl.pallas_call(
        body,
        grid=(10,),
        in_specs=[
            pl.BlockSpec((N // 10, F), lambda i: (i, 0)),
            pl.BlockSpec((N // 10, F), lambda i: (i, 0)),
            pl.BlockSpec((F, F), lambda i: (0, 0)),
            pl.BlockSpec((1, F), lambda i: (0, 0)),
        ],
        out_specs=pl.BlockSpec((N // 10, F), lambda i: (i, 0)),
        out_shape=jax.ShapeDtypeStruct((N, F), jnp.float32),
    )(hrp, x, w, b.reshape(1, F))


def _sc_distances(t, src, dst):
    mesh = plsc.VectorSubcoreMesh(core_axis_name="c", subcore_axis_name="s")

    @functools.partial(
        pl.kernel,
        out_type=jax.ShapeDtypeStruct((E,), jnp.float32),
        mesh=mesh,
        compiler_params=pltpu.CompilerParams(needs_layout_passes=False),
        scratch_types=[
            pltpu.VMEM((ECHUNK,), jnp.int32),
            pltpu.VMEM((ECHUNK,), jnp.int32),
            pltpu.VMEM((ECHUNK, TPAD), jnp.float32),
            pltpu.VMEM((ECHUNK, TPAD), jnp.float32),
            pltpu.VMEM((ECHUNK,), jnp.int32),
            pltpu.VMEM((ECHUNK,), jnp.int32),
            pltpu.VMEM((ECHUNK, TPAD), jnp.float32),
            pltpu.VMEM((ECHUNK, TPAD), jnp.float32),
            pltpu.VMEM((ECHUNK,), jnp.float32),
            pltpu.SemaphoreType.DMA,
            pltpu.SemaphoreType.DMA,
            pltpu.SemaphoreType.DMA,
            pltpu.SemaphoreType.DMA,
        ],
    )
    def kern(t_hbm, src_hbm, dst_hbm, d_hbm,
             sidx0, didx0, ta0, tb0, sidx1, didx1, ta1, tb1, db,
             sa0, sb0, sa1, sb1):
        wid = lax.axis_index("s") * NC + lax.axis_index("c")
        nci = (NCHUNKS - wid + NW - 1) // NW
        lanes = lax.iota(jnp.int32, 16)
        bufs = ((sidx0, didx0, ta0, tb0, sa0, sb0),
                (sidx1, didx1, ta1, tb1, sa1, sb1))

        def fire(ci, b):
            sidx, didx, ta, tb, sa, sb = b
            e0 = pl.multiple_of((wid + ci * NW) * ECHUNK, ECHUNK)
            pltpu.sync_copy(src_hbm.at[pl.ds(e0, ECHUNK)], sidx)
            pltpu.sync_copy(dst_hbm.at[pl.ds(e0, ECHUNK)], didx)
            pltpu.async_copy(t_hbm.at[sidx], ta, sa)
            pltpu.async_copy(t_hbm.at[didx], tb, sb)

        def compute(ci, b):
            sidx, didx, ta, tb, sa, sb = b
            e0 = pl.multiple_of((wid + ci * NW) * ECHUNK, ECHUNK)
            pltpu.make_async_copy(t_hbm.at[sidx], ta, sa).wait()
            pltpu.make_async_copy(t_hbm.at[didx], tb, sb).wait()

            def grp_body(gi, _):
                dvec = jnp.zeros((16,), jnp.float32)
                for i in range(16):
                    row = gi * 16 + i
                    acc = jnp.zeros((16,), jnp.float32)
                    for q in range(TCLS // 16):
                        a = ta[row, pl.ds(q * 16, 16)]
                        bb = tb[row, pl.ds(q * 16, 16)]
                        acc = acc + jnp.abs(a - bb)
                    dvec = jnp.where(lanes == i, jnp.sum(acc), dvec)
                db[pl.ds(gi * 16, 16)] = dvec
                return 0

            lax.fori_loop(0, ECHUNK // 16, grp_body, 0)
            pltpu.sync_copy(db, d_hbm.at[pl.ds(e0, ECHUNK)])

        fire(0, bufs[0])

        def pair_body(p, _):
            c0 = 2 * p
            c1 = 2 * p + 1
            pl.when(c1 < nci)(lambda: fire(c1, bufs[1]))
            compute(c0, bufs[0])
            pl.when(c1 + 1 < nci)(lambda: fire(c1 + 1, bufs[0]))
            pl.when(c1 < nci)(lambda: compute(c1, bufs[1]))
            return 0

        lax.fori_loop(0, (nci + 1) // 2, pair_body, 0, unroll=False)

    return kern(t, src, dst)


def _sc_select_aggregate(d, src, dst, x, bnd):
    mesh = plsc.VectorSubcoreMesh(core_axis_name="c", subcore_axis_name="s")

    @functools.partial(
        pl.kernel,
        out_type=jax.ShapeDtypeStruct((N, F), jnp.float32),
        mesh=mesh,
        compiler_params=pltpu.CompilerParams(needs_layout_passes=False),
        scratch_types=[
            pltpu.VMEM((48,), jnp.int32),
            pltpu.VMEM((CAP + 16,), jnp.float32),
            pltpu.VMEM((CAP + 16,), jnp.int32),
            pltpu.VMEM((CAP + 16,), jnp.int32),
            pltpu.VMEM((144,), jnp.int32),
            pltpu.VMEM((144,), jnp.int32),
            pltpu.VMEM((144,), jnp.float32),
            pltpu.VMEM((128, F), jnp.float32),
            pltpu.VMEM((16, F), jnp.float32),
            pltpu.SemaphoreType.DMA,
        ],
    )
    def kern(d_hbm, src_hbm, dst_hbm, x_hbm, bnd_hbm, hr_hbm,
             bnd_v, dw, srcw, dstw, pidx, pslot, pscale, xrg, outb, sem):
        wid = lax.axis_index("s") * NC + lax.axis_index("c")
        pltpu.sync_copy(bnd_hbm, bnd_v)
        bv = bnd_v[pl.ds(wid, 16)]
        e_lo = bv[0]
        e_hi = bv[1]
        n_lo = wid * NPW
        n_hi = jnp.minimum(n_lo + NPW, N)
        lanes = lax.iota(jnp.int32, 16)
        zf16 = jnp.zeros((16,), jnp.float32)
        zi16 = jnp.zeros((16,), jnp.int32)

        def reload(base):
            base = pl.multiple_of(base, 16)
            pltpu.sync_copy(d_hbm.at[pl.ds(base, CAP)], dw.at[pl.ds(0, CAP)])
            pltpu.sync_copy(src_hbm.at[pl.ds(base, CAP)],
                            srcw.at[pl.ds(0, CAP)])
            pltpu.sync_copy(dst_hbm.at[pl.ds(base, CAP)],
                            dstw.at[pl.ds(0, CAP)])

        def ensure(win, lo, hi):
            need = jnp.logical_or(lo < win, hi > win + CAP)
            new_win = jnp.where(
                need, jnp.clip((lo // 16) * 16, 0, E - CAP), win)
            pl.when(need)(lambda: reload(new_win))
            return new_win

        win0 = jnp.clip((e_lo // 16) * 16, 0, E - CAP)
        reload(win0)
        for q in range(9):
            pidx[pl.ds(q * 16, 16)] = zi16

        def scal(v):
            return v[0] if getattr(v, "ndim", 0) == 1 else v

        def drain(m):
            pltpu.async_copy(x_hbm.at[pidx.at[pl.ds(0, 128)]], xrg, sem).wait()

            def row_add(r, _):
                sl = pslot[pl.ds(r, 16)][0]
                scv = jnp.broadcast_to(pscale[pl.ds(r, 16)][0], (16,))
                for q in range(F // 16):
                    plsc.addupdate(outb.at[sl, pl.ds(q * 16, 16)],
                                   xrg[r, pl.ds(q * 16, 16)] * scv)
                return 0

            lax.fori_loop(0, m, row_add, 0)

        def node_body(n, carry):
            e_ptr, win, np_ = carry
            slot = (n - n_lo) % 16

            def zero_outb():
                for r in range(16):
                    for q in range(F // 16):
                        outb[r, pl.ds(q * 16, 16)] = zf16

            pl.when(slot == 0)(zero_outb)

            def scan_cond(st):
                return jnp.logical_not(st[2])

            def scan_step(st):
                e, w, _ = st
                w = ensure(w, e, jnp.minimum(e + 16, E))
                dvec = dstw[pl.ds(e - w, 16)]
                posv = e + lanes
                stop = jnp.logical_or(
                    jnp.logical_or(dvec != n, posv >= e_hi), lanes == 15)
                adv = jnp.clip(scal(plsc.all_reduce_ffs(stop)), 0, 15)
                return (e + adv, w, adv < 15)

            e_end, win, _ = lax.while_loop(
                scan_cond, scan_step, (e_ptr, win, e_ptr >= e_hi))
            s = e_ptr
            g = e_end - s
            k = (g + 1) // 2

            def seg_case():
                w2 = ensure(win, s, e_end)
                t0_base = (s // 16) * 16
                n_tch = (e_end - t0_base + 15) // 16
                scale_v = jnp.full((16,), P_FRAC, jnp.float32) / \
                    jnp.broadcast_to(k.astype(jnp.float32), (16,))
                slot_v = jnp.broadcast_to(slot, (16,))

                def tgt_body(c, np_c):
                    t0 = t0_base + c * 16
                    off = t0 - w2
                    tgt_d = dw[pl.ds(off, 16)]
                    pos = t0 + lanes
                    valid = jnp.logical_and(pos >= s, pos < e_end)

                    def src_chunk(sc, cnt):
                        sbase = t0_base + sc * 16
                        sv = dw[pl.ds(sbase - w2, 16)]
                        for l in range(16):
                            jpos = sbase + l
                            okj = jnp.logical_and(jpos >= s, jpos < e_end)
                            djv = jnp.broadcast_to(sv[l], (16,))
                            less = djv < tgt_d
                            eqb = jnp.logical_and(djv == tgt_d, jpos < pos)
                            cond = jnp.logical_and(
                                jnp.logical_or(less, eqb), okj)
                            cnt = cnt + jnp.where(cond, 1, 0).astype(jnp.int32)
                        return cnt

                    cnt = lax.fori_loop(0, n_tch, src_chunk, zi16)
                    sel = jnp.logical_and(cnt < k, valid)
                    pc = plsc.all_reduce_population_count(sel)[0]
                    pl.when(np_c > 112)(lambda: drain(np_c))
                    np_c = jnp.where(np_c > 112, 0, np_c)
                    srcvec = srcw[pl.ds(off, 16)]
                    plsc.store_compressed(
                        pidx.at[pl.ds(np_c, 16)], srcvec, mask=sel)
                    plsc.store_compressed(
                        pslot.at[pl.ds(np_c, 16)], slot_v, mask=sel)
                    plsc.store_compressed(
                        pscale.at[pl.ds(np_c, 16)], scale_v, mask=sel)
                    return np_c + pc

                np2 = lax.fori_loop(0, n_tch, tgt_body, np_)
                return (w2, np2)

            win, np_ = lax.cond(g == 0, lambda: (win, np_), seg_case)

            def out_flush():
                pl.when(np_ > 0)(lambda: drain(np_))
                nb = pl.multiple_of(n - 15, 16)
                pltpu.sync_copy(outb, hr_hbm.at[pl.ds(nb, 16)])

            pl.when(slot == 15)(out_flush)
            np_ = jnp.where(slot == 15, 0, np_)
            return (e_end, win, np_)

        lax.fori_loop(n_lo, n_hi, node_body, (e_lo, win0, jnp.int32(0)),
                      unroll=False)

    return kern(d, src, dst, x, bnd)


def kernel(x, src, dst, W_mlp, b_mlp, W_lin, b_lin):
    w1 = jnp.pad(W_mlp, ((0, 0), (0, TPAD - TCLS)))
    b1 = jnp.pad(b_mlp, (0, TPAD - TCLS))
    t = _tc_matmul1(x, w1, b1)
    d = _sc_distances(t, src, dst)
    node_b = jnp.minimum(jnp.arange(NW + 1, dtype=jnp.int32) * NPW, N)
    bnd = jnp.searchsorted(dst, node_b, side="left").astype(jnp.int32)
    bnd = jnp.pad(bnd, (0, 48 - (NW + 1)), constant_values=E)
    hrp = _sc_select_aggregate(d, src, dst, x, bnd)
    return _tc_matmul2(hrp, x, W_lin, b_lin)


# speedup vs baseline: 1.0583x; 1.0583x over previous
import functools

import jax
import jax.numpy as jnp
from jax import lax
from jax.experimental import pallas as pl
from jax.experimental.pallas import tpu as pltpu
from jax.experimental.pallas import tpu_sc as plsc

N = 10000
E = 160000
F = 256
TCLS = 64
TPAD = 128
P_FRAC = 0.5

NC = 2
NS = 16
NW = NC * NS
NPW = 320

ECHUNK = 128
NCHUNKS = E // ECHUNK
CAP = 8192
GMAX = CAP - 16


def _tc_matmul1(x, w, b):
    def body(x_ref, w_ref, b_ref, o_ref):
        o_ref[...] = jnp.tanh(
            jnp.dot(x_ref[...], w_ref[...], preferred_element_type=jnp.float32)
            + b_ref[...]
        )

    return pl.pallas_call(
        body,
        grid=(10,),
        in_specs=[
            pl.BlockSpec((N // 10, 256), lambda i: (i, 0)),
            pl.BlockSpec((256, TPAD), lambda i: (0, 0)),
            pl.BlockSpec((1, TPAD), lambda i: (0, 0)),
        ],
        out_specs=pl.BlockSpec((N // 10, TPAD), lambda i: (i, 0)),
        out_shape=jax.ShapeDtypeStruct((N, TPAD), jnp.float32),
    )(x, w, b.reshape(1, TPAD))


def _tc_matmul2(hrp, x, w, b):
    def body(h_ref, x_ref, w_ref, b_ref, o_ref):
        o_ref[...] = (
            jnp.dot(h_ref[...] + x_ref[...], w_ref[...],
                    preferred_element_type=jnp.float32)
            + b_ref[...]
        )

    return pl.pallas_call(
        body,
        grid=(10,),
        in_specs=[
            pl.BlockSpec((N // 10, F), lambda i: (i, 0)),
            pl.BlockSpec((N // 10, F), lambda i: (i, 0)),
            pl.BlockSpec((F, F), lambda i: (0, 0)),
            pl.BlockSpec((1, F), lambda i: (0, 0)),
        ],
        out_specs=pl.BlockSpec((N // 10, F), lambda i: (i, 0)),
        out_shape=jax.ShapeDtypeStruct((N, F), jnp.float32),
    )(hrp, x, w, b.reshape(1, F))


def _sc_distances(t, src, dst):
    mesh = plsc.VectorSubcoreMesh(core_axis_name="c", subcore_axis_name="s")

    @functools.partial(
        pl.kernel,
        out_type=jax.ShapeDtypeStruct((E,), jnp.float32),
        mesh=mesh,
        compiler_params=pltpu.CompilerParams(needs_layout_passes=False),
        scratch_types=[
            pltpu.VMEM((ECHUNK,), jnp.int32),
            pltpu.VMEM((ECHUNK,), jnp.int32),
            pltpu.VMEM((ECHUNK, TPAD), jnp.float32),
            pltpu.VMEM((ECHUNK, TPAD), jnp.float32),
            pltpu.VMEM((ECHUNK,), jnp.int32),
            pltpu.VMEM((ECHUNK,), jnp.int32),
            pltpu.VMEM((ECHUNK, TPAD), jnp.float32),
            pltpu.VMEM((ECHUNK, TPAD), jnp.float32),
            pltpu.VMEM((ECHUNK,), jnp.float32),
            pltpu.SemaphoreType.DMA,
            pltpu.SemaphoreType.DMA,
            pltpu.SemaphoreType.DMA,
            pltpu.SemaphoreType.DMA,
        ],
    )
    def kern(t_hbm, src_hbm, dst_hbm, d_hbm,
             sidx0, didx0, ta0, tb0, sidx1, didx1, ta1, tb1, db,
             sa0, sb0, sa1, sb1):
        wid = lax.axis_index("s") * NC + lax.axis_index("c")
        nci = (NCHUNKS - wid + NW - 1) // NW
        lanes = lax.iota(jnp.int32, 16)
        bufs = ((sidx0, didx0, ta0, tb0, sa0, sb0),
                (sidx1, didx1, ta1, tb1, sa1, sb1))

        def fire(ci, b):
            sidx, didx, ta, tb, sa, sb = b
            e0 = pl.multiple_of((wid + ci * NW) * ECHUNK, ECHUNK)
            pltpu.sync_copy(src_hbm.at[pl.ds(e0, ECHUNK)], sidx)
            pltpu.sync_copy(dst_hbm.at[pl.ds(e0, ECHUNK)], didx)
            pltpu.async_copy(t_hbm.at[sidx], ta, sa)
            pltpu.async_copy(t_hbm.at[didx], tb, sb)

        def compute(ci, b):
            sidx, didx, ta, tb, sa, sb = b
            e0 = pl.multiple_of((wid + ci * NW) * ECHUNK, ECHUNK)
            pltpu.make_async_copy(t_hbm.at[sidx], ta, sa).wait()
            pltpu.make_async_copy(t_hbm.at[didx], tb, sb).wait()

            def grp_body(gi, _):
                dvec = jnp.zeros((16,), jnp.float32)
                for i in range(16):
                    row = gi * 16 + i
                    acc = jnp.zeros((16,), jnp.float32)
                    for q in range(TCLS // 16):
                        a = ta[row, pl.ds(q * 16, 16)]
                        bb = tb[row, pl.ds(q * 16, 16)]
                        acc = acc + jnp.abs(a - bb)
                    dvec = jnp.where(lanes == i, jnp.sum(acc), dvec)
                db[pl.ds(gi * 16, 16)] = dvec
                return 0

            lax.fori_loop(0, ECHUNK // 16, grp_body, 0)
            pltpu.sync_copy(db, d_hbm.at[pl.ds(e0, ECHUNK)])

        fire(0, bufs[0])

        def pair_body(p, _):
            c0 = 2 * p
            c1 = 2 * p + 1
            pl.when(c1 < nci)(lambda: fire(c1, bufs[1]))
            compute(c0, bufs[0])
            pl.when(c1 + 1 < nci)(lambda: fire(c1 + 1, bufs[0]))
            pl.when(c1 < nci)(lambda: compute(c1, bufs[1]))
            return 0

        lax.fori_loop(0, (nci + 1) // 2, pair_body, 0, unroll=False)

    return kern(t, src, dst)


def _sc_select_aggregate(d, src, dst, x, bnd):
    mesh = plsc.VectorSubcoreMesh(core_axis_name="c", subcore_axis_name="s")

    @functools.partial(
        pl.kernel,
        out_type=jax.ShapeDtypeStruct((N, F), jnp.float32),
        mesh=mesh,
        compiler_params=pltpu.CompilerParams(needs_layout_passes=False),
        scratch_types=[
            pltpu.VMEM((48,), jnp.int32),
            pltpu.VMEM((CAP + 16,), jnp.float32),
            pltpu.VMEM((CAP + 16,), jnp.int32),
            pltpu.VMEM((CAP + 16,), jnp.int32),
            pltpu.VMEM((144,), jnp.int32),
            pltpu.VMEM((144,), jnp.int32),
            pltpu.VMEM((144,), jnp.float32),
            pltpu.VMEM((128, F), jnp.float32),
            pltpu.VMEM((16, F), jnp.float32),
            pltpu.SemaphoreType.DMA,
        ],
    )
    def kern(d_hbm, src_hbm, dst_hbm, x_hbm, bnd_hbm, hr_hbm,
             bnd_v, dw, srcw, dstw, pidx, pslot, pscale, xrg, outb, sem):
        wid = lax.axis_index("s") * NC + lax.axis_index("c")
        pltpu.sync_copy(bnd_hbm, bnd_v)
        bv = bnd_v[pl.ds(wid, 16)]
        e_lo = bv[0]
        e_hi = bv[1]
        n_lo = wid * NPW
        n_hi = jnp.minimum(n_lo + NPW, N)
        lanes = lax.iota(jnp.int32, 16)
        zf16 = jnp.zeros((16,), jnp.float32)
        zi16 = jnp.zeros((16,), jnp.int32)

        def reload(base):
            base = pl.multiple_of(base, 16)
            pltpu.sync_copy(d_hbm.at[pl.ds(base, CAP)], dw.at[pl.ds(0, CAP)])
            pltpu.sync_copy(src_hbm.at[pl.ds(base, CAP)],
                            srcw.at[pl.ds(0, CAP)])
            pltpu.sync_copy(dst_hbm.at[pl.ds(base, CAP)],
                            dstw.at[pl.ds(0, CAP)])

        def ensure(win, lo, hi):
            need = jnp.logical_or(lo < win, hi > win + CAP)
            new_win = jnp.where(
                need, jnp.clip((lo // 16) * 16, 0, E - CAP), win)
            pl.when(need)(lambda: reload(new_win))
            return new_win

        win0 = jnp.clip((e_lo // 16) * 16, 0, E - CAP)
        reload(win0)
        for q in range(9):
            pidx[pl.ds(q * 16, 16)] = zi16

        def scal(v):
            return v[0] if getattr(v, "ndim", 0) == 1 else v

        def drain(m):
            pltpu.async_copy(x_hbm.at[pidx.at[pl.ds(0, 128)]], xrg, sem).wait()

            def row_add(r, _):
                sl = pslot[pl.ds(r, 16)][0]
                scv = jnp.broadcast_to(pscale[pl.ds(r, 16)][0], (16,))
                for q in range(F // 16):
                    plsc.addupdate(outb.at[sl, pl.ds(q * 16, 16)],
                                   xrg[r, pl.ds(q * 16, 16)] * scv)
                return 0

            lax.fori_loop(0, m, row_add, 0)

        def node_body(n, carry):
            e_ptr, win, np_ = carry
            slot = (n - n_lo) % 16

            def zero_outb():
                for r in range(16):
                    for q in range(F // 16):
                        outb[r, pl.ds(q * 16, 16)] = zf16

            pl.when(slot == 0)(zero_outb)

            def scan_cond(st):
                return jnp.logical_not(st[2])

            def scan_step(st):
                e, w, _ = st
                w = ensure(w, e, jnp.minimum(e + 16, E))
                dvec = dstw[pl.ds(e - w, 16)]
                posv = e + lanes
                stop = jnp.logical_or(
                    jnp.logical_or(dvec != n, posv >= e_hi), lanes == 15)
                adv = jnp.clip(scal(plsc.all_reduce_ffs(stop)), 0, 15)
                return (e + adv, w, adv < 15)

            e_end, win, _ = lax.while_loop(
                scan_cond, scan_step, (e_ptr, win, e_ptr >= e_hi))
            s = e_ptr
            g = e_end - s
            k = (g + 1) // 2

            def seg_case():
                w2 = ensure(win, s, e_end)
                t0_base = (s // 16) * 16
                n_tch = (e_end - t0_base + 15) // 16
                scale_v = jnp.full((16,), P_FRAC, jnp.float32) / \
                    jnp.broadcast_to(k.astype(jnp.float32), (16,))
                slot_v = jnp.broadcast_to(slot, (16,))

                def tgt_body(c, np_c):
                    t0 = t0_base + c * 16
                    off = t0 - w2
                    tgt_d = dw[pl.ds(off, 16)]
                    pos = t0 + lanes
                    valid = jnp.logical_and(pos >= s, pos < e_end)

                    def src_body(j, cnt):
                        dj = dw[pl.ds(j - w2, 16)][0]
                        djv = jnp.broadcast_to(dj, (16,))
                        less = djv < tgt_d
                        eqb = jnp.logical_and(djv == tgt_d, j < pos)
                        return cnt + jnp.where(
                            jnp.logical_or(less, eqb), 1, 0).astype(jnp.int32)

                    cnt = lax.fori_loop(s, e_end, src_body, zi16)
                    sel = jnp.logical_and(cnt < k, valid)
                    pc = plsc.all_reduce_population_count(sel)[0]
                    pl.when(np_c > 112)(lambda: drain(np_c))
                    np_c = jnp.where(np_c > 112, 0, np_c)
                    srcvec = srcw[pl.ds(off, 16)]
                    plsc.store_compressed(
                        pidx.at[pl.ds(np_c, 16)], srcvec, mask=sel)
                    plsc.store_compressed(
                        pslot.at[pl.ds(np_c, 16)], slot_v, mask=sel)
                    plsc.store_compressed(
                        pscale.at[pl.ds(np_c, 16)], scale_v, mask=sel)
                    return np_c + pc

                np2 = lax.fori_loop(0, n_tch, tgt_body, np_)
                return (w2, np2)

            win, np_ = lax.cond(g == 0, lambda: (win, np_), seg_case)

            def out_flush():
                pl.when(np_ > 0)(lambda: drain(np_))
                nb = pl.multiple_of(n - 15, 16)
                pltpu.sync_copy(outb, hr_hbm.at[pl.ds(nb, 16)])

            pl.when(slot == 15)(out_flush)
            np_ = jnp.where(slot == 15, 0, np_)
            return (e_end, win, np_)

        lax.fori_loop(n_lo, n_hi, node_body, (e_lo, win0, jnp.int32(0)),
                      unroll=False)

    return kern(d, src, dst, x, bnd)


def kernel(x, src, dst, W_mlp, b_mlp, W_lin, b_lin):
    w1 = jnp.pad(W_mlp, ((0, 0), (0, TPAD - TCLS)))
    b1 = jnp.pad(b_mlp, (0, TPAD - TCLS))
    t = _tc_matmul1(x, w1, b1)
    d = _sc_distances(t, src, dst)
    node_b = jnp.minimum(jnp.arange(NW + 1, dtype=jnp.int32) * NPW, N)
    bnd = jnp.searchsorted(dst, node_b, side="left").astype(jnp.int32)
    bnd = jnp.pad(bnd, (0, 48 - (NW + 1)), constant_values=E)
    hrp = _sc_select_aggregate(d, src, dst, x, bnd)
    return _tc_matmul2(hrp, x, W_lin, b_lin)
